# Initial kernel scaffold; baseline (speedup 1.0000x reference)
import jax
import jax.numpy as jnp
from jax.experimental import pallas as pl


def kernel(means, vertices):
    raise NotImplementedError("write your pallas kernel here")


# speedup vs baseline: 3.1990x; 3.1990x over previous
import functools

import jax
import jax.numpy as jnp
from jax import lax
from jax.experimental import pallas as pl

Q = 50000
V = 5023
VP = 5120
B = 256
QP = ((Q + B - 1) // B) * B
K = 8
PAD_COORD = 1e18


def _knn_body(m_ref, vt_ref, out_ref):
    m = m_ref[...]
    vt = vt_ref[...]
    vx = vt[0:1, :]
    vy = vt[1:2, :]
    vz = vt[2:3, :]
    mx = m[:, 0:1]
    my = m[:, 1:2]
    mz = m[:, 2:3]
    m2 = mx * mx + my * my + mz * mz
    v2 = vx * vx + vy * vy + vz * vz
    dot = jax.lax.dot_general(
        m, vt, (((1,), (0,)), ((), ())),
        preferred_element_type=jnp.float32)
    d2 = m2 - 2.0 * dot + v2

    iota = lax.broadcasted_iota(jnp.int32, (B, VP), 1)
    inf = jnp.float32(jnp.inf)
    for j in range(K):
        mval = jnp.min(d2, axis=1, keepdims=True)
        cand = jnp.where(d2 == mval, iota, VP)
        idx = jnp.min(cand, axis=1, keepdims=True)
        out_ref[:, j:j + 1] = idx
        d2 = jnp.where(cand == idx, inf, d2)


@jax.jit
def kernel(means, vertices):
    mp = jnp.zeros((QP, 8), jnp.float32).at[:Q, :3].set(means)
    vt = jnp.full((8, VP), 0.0, jnp.float32)
    vt = vt.at[:3, :V].set(vertices.T)
    vt = vt.at[:3, V:].set(PAD_COORD)

    grid = (QP // B,)
    out = pl.pallas_call(
        _knn_body,
        grid=grid,
        in_specs=[
            pl.BlockSpec((B, 8), lambda i: (i, 0)),
            pl.BlockSpec((8, VP), lambda i: (0, 0)),
        ],
        out_specs=pl.BlockSpec((B, K), lambda i: (i, 0)),
        out_shape=jax.ShapeDtypeStruct((QP, K), jnp.int32),
    )(mp, vt)
    index_cache = out[:Q]
    hit_rate = jnp.array(0.0, dtype=jnp.float32)
    return index_cache, hit_rate


# speedup vs baseline: 3.2034x; 1.0014x over previous
import functools

import jax
import jax.numpy as jnp
from jax import lax
from jax.experimental import pallas as pl

Q = 50000
V = 5023
VP = 5120
B = 256
QP = ((Q + B - 1) // B) * B
K = 8
PAD_COORD = 1e18


def _knn_body(m_ref, vt_ref, out_ref):
    m = m_ref[...]
    vt = vt_ref[...]
    vx = vt[0:1, :]
    vy = vt[1:2, :]
    vz = vt[2:3, :]
    mx = m[:, 0:1]
    my = m[:, 1:2]
    mz = m[:, 2:3]
    m2 = mx * mx + my * my + mz * mz
    v2 = vx * vx + vy * vy + vz * vz
    dot = jnp.dot(m, vt)
    d2 = m2 - 2.0 * dot + v2

    iota = lax.broadcasted_iota(jnp.int32, (B, VP), 1)
    inf = jnp.float32(jnp.inf)
    for j in range(K):
        mval = jnp.min(d2, axis=1, keepdims=True)
        cand = jnp.where(d2 == mval, iota, VP)
        idx = jnp.min(cand, axis=1, keepdims=True)
        out_ref[:, j:j + 1] = idx
        d2 = jnp.where(cand == idx, inf, d2)


@jax.jit
def kernel(means, vertices):
    mp = jnp.zeros((QP, 8), jnp.float32).at[:Q, :3].set(means)
    vt = jnp.full((8, VP), 0.0, jnp.float32)
    vt = vt.at[:3, :V].set(vertices.T)
    vt = vt.at[:3, V:].set(PAD_COORD)

    grid = (QP // B,)
    out = pl.pallas_call(
        _knn_body,
        grid=grid,
        in_specs=[
            pl.BlockSpec((B, 8), lambda i: (i, 0)),
            pl.BlockSpec((8, VP), lambda i: (0, 0)),
        ],
        out_specs=pl.BlockSpec((B, K), lambda i: (i, 0)),
        out_shape=jax.ShapeDtypeStruct((QP, K), jnp.int32),
    )(mp, vt)
    index_cache = out[:Q]
    hit_rate = jnp.array(0.0, dtype=jnp.float32)
    return index_cache, hit_rate
